# Initial kernel scaffold; baseline (speedup 1.0000x reference)
import jax
import jax.numpy as jnp
from jax.experimental import pallas as pl


def kernel(node_feat, edge_attr, edge_index):
    raise NotImplementedError("write your pallas kernel here")


# speedup vs baseline: 2.3639x; 2.3639x over previous
import functools

import jax
import jax.numpy as jnp
from jax import lax
from jax.experimental import pallas as pl
from jax.experimental.pallas import tpu as pltpu
from jax.experimental.pallas import tpu_sc as plsc

EPS = 1e-08

_NC = 2
_NS = 16
_NW = _NC * _NS

_CH = 80


def _sc_gather(idx_flat, table):
    r_total = idx_flat.shape[0]
    v, d = table.shape
    rows_per_w = r_total // _NW
    chunks_per_w = rows_per_w // _CH

    mesh = plsc.VectorSubcoreMesh(
        core_axis_name="c", subcore_axis_name="s",
        num_cores=_NC, num_subcores=_NS)

    @functools.partial(
        pl.kernel,
        mesh=mesh,
        out_type=jax.ShapeDtypeStruct((r_total, d), jnp.float32),
        scratch_types=[
            pltpu.VMEM((rows_per_w,), jnp.int32),
            pltpu.VMEM((_CH, d), jnp.float32),
            pltpu.SemaphoreType.DMA,
        ],
    )
    def gather_kernel(idx_hbm, table_hbm, out_hbm, idx_v, rows_v, sem):
        wid = lax.axis_index("s") * _NC + lax.axis_index("c")
        row0 = wid * rows_per_w
        pltpu.sync_copy(idx_hbm.at[pl.ds(row0, rows_per_w)], idx_v)

        def body(c, carry):
            idx_c = idx_v.at[pl.ds(c * _CH, _CH)]
            pltpu.async_copy(table_hbm.at[idx_c], rows_v, sem).wait()
            pltpu.sync_copy(rows_v, out_hbm.at[pl.ds(row0 + c * _CH, _CH)])
            return carry

        lax.fori_loop(0, chunks_per_w, body, 0, unroll=False)

    return gather_kernel(idx_flat, table)


def _tc_expand_body(dst_ref, src_ref, ea_ref, out_ref):
    src = src_ref[...]
    dst = dst_ref[...]
    ea = ea_ref[...]
    a = ea.shape[1]
    d = src.shape[1]
    diff = src - dst
    nsq = jnp.sum(diff * diff, axis=1, keepdims=True)
    norm = jnp.sqrt(nsq)
    denom = norm + EPS
    inv = 1.0 / denom
    unit = diff * inv
    ea_inv = 1.0 / (ea + EPS)
    o = 0
    out_ref[:, o:o + a] = ea; o += a
    out_ref[:, o:o + d] = src; o += d
    out_ref[:, o:o + d] = dst; o += d
    out_ref[:, o:o + d] = diff; o += d
    out_ref[:, o:o + d] = unit; o += d
    out_ref[:, o:o + 1] = norm; o += 1
    out_ref[:, o:o + 1] = inv; o += 1
    out_ref[:, o:o + 1] = nsq; o += 1
    out_ref[:, o:o + 1] = inv * inv; o += 1
    out_ref[:, o:o + a] = ea_inv; o += a
    out_ref[:, o:o + a] = ea * ea; o += a
    out_ref[:, o:o + a] = ea_inv * ea_inv; o += a


def _tc_expand(gathered, edge_attr, block_rows=512):
    r, d = gathered.shape
    e, a = edge_attr.shape
    width = a * 4 + d * 4 + 4
    n_blocks = e // block_rows
    grid = (n_blocks,)
    return pl.pallas_call(
        _tc_expand_body,
        grid=grid,
        in_specs=[
            pl.BlockSpec((block_rows, d), lambda i: (i, 0)),
            pl.BlockSpec((block_rows, d), lambda i: (n_blocks + i, 0)),
            pl.BlockSpec((block_rows, a), lambda i: (i, 0)),
        ],
        out_specs=pl.BlockSpec((block_rows, width), lambda i: (i, 0)),
        out_shape=jax.ShapeDtypeStruct((e, width), jnp.float32),
        compiler_params=pltpu.CompilerParams(
            dimension_semantics=("arbitrary",),
        ),
    )(gathered, gathered, edge_attr)


def kernel(node_feat, edge_attr, edge_index):
    e = edge_index.shape[1]
    idx_flat = edge_index.reshape(-1)
    gathered = _sc_gather(idx_flat, node_feat)
    return _tc_expand(gathered, edge_attr)


# speedup vs baseline: 2.5646x; 1.0849x over previous
import functools

import jax
import jax.numpy as jnp
from jax import lax
from jax.experimental import pallas as pl
from jax.experimental.pallas import tpu as pltpu
from jax.experimental.pallas import tpu_sc as plsc

EPS = 1e-08

_NC = 2
_NS = 16
_NW = _NC * _NS

_CH = 80


def _sc_gather(idx_flat, table):
    r_total = idx_flat.shape[0]
    v, d = table.shape
    rows_per_w = r_total // _NW
    chunks_per_w = rows_per_w // _CH

    mesh = plsc.VectorSubcoreMesh(
        core_axis_name="c", subcore_axis_name="s",
        num_cores=_NC, num_subcores=_NS)

    nbuf = 4
    look = 2

    @functools.partial(
        pl.kernel,
        mesh=mesh,
        out_type=jax.ShapeDtypeStruct((r_total, d), jnp.float32),
        scratch_types=[
            pltpu.VMEM((rows_per_w,), jnp.int32),
            pltpu.VMEM((nbuf, _CH, d), jnp.float32),
            pltpu.SemaphoreType.DMA,
            pltpu.SemaphoreType.DMA,
        ],
    )
    def gather_kernel(idx_hbm, table_hbm, out_hbm, idx_v, rows_v, sem_g,
                      sem_w):
        wid = lax.axis_index("s") * _NC + lax.axis_index("c")
        row0 = wid * rows_per_w
        pltpu.sync_copy(idx_hbm.at[pl.ds(row0, rows_per_w)], idx_v)

        def start_g(c):
            idx_c = idx_v.at[pl.ds(c * _CH, _CH)]
            pltpu.async_copy(table_hbm.at[idx_c], rows_v.at[c % nbuf], sem_g)

        def wait_g(c):
            idx_c = idx_v.at[pl.ds(c * _CH, _CH)]
            pltpu.make_async_copy(
                table_hbm.at[idx_c], rows_v.at[c % nbuf], sem_g).wait()

        def start_w(c):
            pltpu.async_copy(rows_v.at[c % nbuf],
                             out_hbm.at[pl.ds(row0 + c * _CH, _CH)], sem_w)

        def wait_w(c):
            pltpu.make_async_copy(
                rows_v.at[c % nbuf],
                out_hbm.at[pl.ds(row0 + c * _CH, _CH)], sem_w).wait()

        for c in range(look):
            start_g(c)

        def body(c, carry):
            wait_g(c)
            start_w(c)

            @pl.when(c + look < chunks_per_w)
            def _():
                @pl.when(c >= look)
                def _():
                    wait_w(c - look)
                start_g(c + look)

            return carry

        lax.fori_loop(0, chunks_per_w, body, 0, unroll=False)
        for c in range(chunks_per_w - nbuf, chunks_per_w):
            wait_w(c)

    return gather_kernel(idx_flat, table)


def _tc_expand_body(dst_ref, src_ref, ea_ref, out_ref):
    src = src_ref[...]
    dst = dst_ref[...]
    ea = ea_ref[...]
    a = ea.shape[1]
    d = src.shape[1]
    diff = src - dst
    nsq = jnp.sum(diff * diff, axis=1, keepdims=True)
    norm = jnp.sqrt(nsq)
    denom = norm + EPS
    inv = 1.0 / denom
    unit = diff * inv
    ea_inv = 1.0 / (ea + EPS)
    o = 0
    out_ref[:, o:o + a] = ea; o += a
    out_ref[:, o:o + d] = src; o += d
    out_ref[:, o:o + d] = dst; o += d
    out_ref[:, o:o + d] = diff; o += d
    out_ref[:, o:o + d] = unit; o += d
    out_ref[:, o:o + 1] = norm; o += 1
    out_ref[:, o:o + 1] = inv; o += 1
    out_ref[:, o:o + 1] = nsq; o += 1
    out_ref[:, o:o + 1] = inv * inv; o += 1
    out_ref[:, o:o + a] = ea_inv; o += a
    out_ref[:, o:o + a] = ea * ea; o += a
    out_ref[:, o:o + a] = ea_inv * ea_inv; o += a


def _tc_expand(gathered, edge_attr, block_rows=512):
    r, d = gathered.shape
    e, a = edge_attr.shape
    width = a * 4 + d * 4 + 4
    n_blocks = e // block_rows
    grid = (n_blocks,)
    return pl.pallas_call(
        _tc_expand_body,
        grid=grid,
        in_specs=[
            pl.BlockSpec((block_rows, d), lambda i: (i, 0)),
            pl.BlockSpec((block_rows, d), lambda i: (n_blocks + i, 0)),
            pl.BlockSpec((block_rows, a), lambda i: (i, 0)),
        ],
        out_specs=pl.BlockSpec((block_rows, width), lambda i: (i, 0)),
        out_shape=jax.ShapeDtypeStruct((e, width), jnp.float32),
        compiler_params=pltpu.CompilerParams(
            dimension_semantics=("arbitrary",),
        ),
    )(gathered, gathered, edge_attr)


def kernel(node_feat, edge_attr, edge_index):
    e = edge_index.shape[1]
    idx_flat = edge_index.reshape(-1)
    gathered = _sc_gather(idx_flat, node_feat)
    return _tc_expand(gathered, edge_attr)
